# baseline (speedup 1.0000x reference)
import functools

import jax
import jax.numpy as jnp
from jax import lax
from jax.experimental import pallas as pl
from jax.experimental.pallas import tpu as pltpu
from jax.experimental.pallas import tpu_sc as plsc

N, E, D, H, L = 10000, 160000, 256, 256, 64
DHALF = D // 2
NC, NS = 2, 16
CHUNK = 112
NBUF = 4
NCHUNK = 92
EPT = NCHUNK * CHUNK
EPAD = EPT * NS
NPAD = 10112
ZROWS = NPAD // NS

_mesh = plsc.VectorSubcoreMesh(core_axis_name="c", subcore_axis_name="s")


@functools.partial(
    pl.kernel,
    mesh=_mesh,
    out_type=[
        jax.ShapeDtypeStruct((NPAD, DHALF), jnp.bfloat16),
        jax.ShapeDtypeStruct((NPAD, DHALF), jnp.bfloat16),
    ],
    scratch_types=[
        pltpu.VMEM((NCHUNK, CHUNK), jnp.int32),
        pltpu.VMEM((NCHUNK, CHUNK), jnp.int32),
        pltpu.VMEM((NBUF, CHUNK, DHALF), jnp.bfloat16),
        pltpu.SemaphoreType.DMA((NBUF,)),
        pltpu.SemaphoreType.DMA((NBUF,)),
        pltpu.VMEM_SHARED((NPAD, DHALF), jnp.bfloat16),
        pltpu.VMEM_SHARED((NPAD, DHALF), jnp.bfloat16),
    ],
    compiler_params=pltpu.CompilerParams(use_tc_tiling_on_sc=False),
)
def _sc_agg(xl_hbm, xr_hbm, src_hbm, dst_hbm, z0_hbm, out0_hbm, out1_hbm,
            src_v, dst_v, rows_v, gsem, ssem, acc_sh, x_sh):
    cid = lax.axis_index("c")
    sid = lax.axis_index("s")

    pltpu.sync_copy(src_hbm.at[sid], src_v)
    pltpu.sync_copy(dst_hbm.at[sid], dst_v)

    def run(xtab, out):
        pltpu.sync_copy(xtab.at[pl.ds(sid * ZROWS, ZROWS)],
                        x_sh.at[pl.ds(sid * ZROWS, ZROWS)])
        pltpu.sync_copy(z0_hbm.at[pl.ds(sid * ZROWS, ZROWS)],
                        acc_sh.at[pl.ds(sid * ZROWS, ZROWS)])
        plsc.subcore_barrier()

        def gather_start(c, b):
            tab = x_sh if b % 2 == 0 else xtab
            pltpu.async_copy(tab.at[src_v.at[c]], rows_v.at[b], gsem.at[b])

        def gather_wait(c, b):
            tab = x_sh if b % 2 == 0 else xtab
            pltpu.make_async_copy(tab.at[src_v.at[c]], rows_v.at[b],
                                  gsem.at[b]).wait()

        def scatter_start(c, b):
            pltpu.async_copy(rows_v.at[b], acc_sh.at[dst_v.at[c]],
                             ssem.at[b], add=True)

        def scatter_wait(c, b):
            pltpu.make_async_copy(rows_v.at[b], acc_sh.at[dst_v.at[c]],
                                  ssem.at[b]).wait()

        for b in range(NBUF):
            gather_start(b, b)

        def body(i, _):
            c0 = i * NBUF
            for b in range(NBUF):
                gather_wait(c0 + b, b)
                scatter_start(c0 + b, b)
            for b in range(NBUF):
                scatter_wait(c0 + b, b)
                nxt = c0 + b + NBUF

                @pl.when(nxt < NCHUNK)
                def _():
                    gather_start(nxt, b)
            return 0
        lax.fori_loop(0, NCHUNK // NBUF, body, 0)
        plsc.subcore_barrier()
        pltpu.sync_copy(acc_sh.at[pl.ds(sid * ZROWS, ZROWS)],
                        out.at[pl.ds(sid * ZROWS, ZROWS)])

    @pl.when(cid == 0)
    def _():
        run(xl_hbm, out0_hbm)

    @pl.when(cid == 1)
    def _():
        run(xr_hbm, out1_hbm)


def _dense_body(x_ref, agg_ref, wm_ref, whx_ref, wha_ref, bh_ref,
                rmw_ref, rmb_ref, rvw_ref, rvb_ref, eps_ref, z_ref):
    agg = agg_ref[...].astype(jnp.float32)
    aggm = jnp.dot(agg, wm_ref[...], preferred_element_type=jnp.float32)
    h = jnp.maximum(
        jnp.dot(x_ref[...], whx_ref[...], preferred_element_type=jnp.float32)
        + jnp.dot(aggm, wha_ref[...], preferred_element_type=jnp.float32)
        + bh_ref[...], 0.0)
    zm = jnp.dot(h, rmw_ref[...], preferred_element_type=jnp.float32) + rmb_ref[...]
    zlv = -jnp.abs(jnp.dot(h, rvw_ref[...], preferred_element_type=jnp.float32)
                   + rvb_ref[...])
    z_ref[...] = zm + jnp.exp(zlv * 0.5) * eps_ref[...]


_ROWS = 2000


def _dense(x, agg, W_m, whx, wha, b_h, R_mean_w, R_mean_b, R_var_w, R_var_b,
           eps):
    grid = (N // _ROWS,)
    full = lambda shape: pl.BlockSpec(shape, lambda i: (0, 0))
    return pl.pallas_call(
        _dense_body,
        grid=grid,
        in_specs=[
            pl.BlockSpec((_ROWS, D), lambda i: (i, 0)),
            pl.BlockSpec((_ROWS, D), lambda i: (i, 0)),
            full((D, H)),
            full((D, H)),
            full((H, H)),
            full((1, H)),
            full((H, L)),
            full((1, L)),
            full((H, L)),
            full((1, L)),
            pl.BlockSpec((_ROWS, L), lambda i: (i, 0)),
        ],
        out_specs=pl.BlockSpec((_ROWS, L), lambda i: (i, 0)),
        out_shape=jax.ShapeDtypeStruct((N, L), jnp.float32),
    )(x, agg, W_m, whx, wha, b_h.reshape(1, H),
      R_mean_w, R_mean_b.reshape(1, L), R_var_w, R_var_b.reshape(1, L), eps)


def kernel(x, edge_index, W_m, W_h, b_h, R_mean_w, R_mean_b, R_var_w, R_var_b):
    src = edge_index[0]
    dst = edge_index[1]
    pad = EPAD - E
    src_p = jnp.concatenate([src, jnp.zeros((pad,), jnp.int32)]
                            ).reshape(NS, NCHUNK, CHUNK)
    dst_p = jnp.concatenate([dst, jnp.full((pad,), N, jnp.int32)]
                            ).reshape(NS, NCHUNK, CHUNK)
    xb = jnp.concatenate(
        [x.astype(jnp.bfloat16),
         jnp.zeros((NPAD - N, D), jnp.bfloat16)])
    xl = xb[:, :DHALF]
    xr = xb[:, DHALF:]
    z0 = jnp.zeros((NPAD, DHALF), jnp.bfloat16)
    aggl, aggr = _sc_agg(xl, xr, src_p, dst_p, z0)
    agg = jnp.concatenate([aggl[:N], aggr[:N]], axis=1)
    eps = jax.random.normal(jax.random.key(1), (N, L), dtype=jnp.float32)
    return _dense(x, agg, W_m, W_h[:D], W_h[D:], b_h,
                  R_mean_w, R_mean_b, R_var_w, R_var_b, eps)


# speedup vs baseline: 1.2281x; 1.2281x over previous
import functools

import jax
import jax.numpy as jnp
from jax import lax
from jax.experimental import pallas as pl
from jax.experimental.pallas import tpu as pltpu
from jax.experimental.pallas import tpu_sc as plsc

N, E, D, H, L = 10000, 160000, 256, 256, 64
DHALF = D // 2
NC, NS = 2, 16
CHUNK = 128
NBUF = 3
NCHUNK = 81
EPT = NCHUNK * CHUNK
EPAD = EPT * NS
NPAD = 10112
ZROWS = NPAD // NS
XLAST = N - (NS - 1) * ZROWS

_mesh = plsc.VectorSubcoreMesh(core_axis_name="c", subcore_axis_name="s")


@functools.partial(
    pl.kernel,
    mesh=_mesh,
    out_type=[
        jax.ShapeDtypeStruct((NPAD, DHALF), jnp.bfloat16),
        jax.ShapeDtypeStruct((NPAD, DHALF), jnp.bfloat16),
    ],
    scratch_types=[
        pltpu.VMEM((NCHUNK, CHUNK), jnp.int32),
        pltpu.VMEM((NCHUNK, CHUNK), jnp.int32),
        pltpu.VMEM((NBUF, CHUNK, DHALF), jnp.bfloat16),
        pltpu.SemaphoreType.DMA((NBUF,)),
        pltpu.SemaphoreType.DMA((NBUF,)),
        pltpu.VMEM_SHARED((NPAD, DHALF), jnp.bfloat16),
        pltpu.VMEM_SHARED((NPAD, DHALF), jnp.bfloat16),
    ],
    compiler_params=pltpu.CompilerParams(use_tc_tiling_on_sc=False),
)
def _sc_agg(xl_hbm, xr_hbm, src_hbm, dst_hbm, z0_hbm, out0_hbm, out1_hbm,
            src_v, dst_v, rows_v, gsem, ssem, acc_sh, x_sh):
    cid = lax.axis_index("c")
    sid = lax.axis_index("s")

    pltpu.sync_copy(src_hbm.at[sid], src_v)
    pltpu.sync_copy(dst_hbm.at[sid], dst_v)

    def run(xtab, out):
        @pl.when(sid < NS - 1)
        def _():
            pltpu.sync_copy(xtab.at[pl.ds(sid * ZROWS, ZROWS)],
                            x_sh.at[pl.ds(sid * ZROWS, ZROWS)])

        @pl.when(sid == NS - 1)
        def _():
            pltpu.sync_copy(xtab.at[pl.ds((NS - 1) * ZROWS, XLAST)],
                            x_sh.at[pl.ds((NS - 1) * ZROWS, XLAST)])

        pltpu.sync_copy(z0_hbm, acc_sh.at[pl.ds(sid * ZROWS, ZROWS)])
        plsc.subcore_barrier()

        def gather_start(c, b):
            pltpu.async_copy(x_sh.at[src_v.at[c]], rows_v.at[b], gsem.at[b])

        def gather_wait(c, b):
            pltpu.make_async_copy(x_sh.at[src_v.at[c]], rows_v.at[b],
                                  gsem.at[b]).wait()

        def scatter_start(c, b):
            pltpu.async_copy(rows_v.at[b], acc_sh.at[dst_v.at[c]],
                             ssem.at[b], add=True)

        def scatter_wait(c, b):
            pltpu.make_async_copy(rows_v.at[b], acc_sh.at[dst_v.at[c]],
                                  ssem.at[b]).wait()

        for b in range(NBUF):
            gather_start(b, b)

        def body(i, _):
            c0 = i * NBUF
            for b in range(NBUF):
                gather_wait(c0 + b, b)
                scatter_start(c0 + b, b)
            for b in range(NBUF):
                scatter_wait(c0 + b, b)
                nxt = c0 + b + NBUF

                @pl.when(nxt < NCHUNK)
                def _():
                    gather_start(nxt, b)
            return 0
        lax.fori_loop(0, NCHUNK // NBUF, body, 0)
        plsc.subcore_barrier()
        pltpu.sync_copy(acc_sh.at[pl.ds(sid * ZROWS, ZROWS)],
                        out.at[pl.ds(sid * ZROWS, ZROWS)])

    @pl.when(cid == 0)
    def _():
        run(xl_hbm, out0_hbm)

    @pl.when(cid == 1)
    def _():
        run(xr_hbm, out1_hbm)


def _dense_body(x_ref, al_ref, ar_ref, wmt_ref, wmb_ref, whx_ref, wha_ref,
                bh_ref, rmw_ref, rmb_ref, rvw_ref, rvb_ref, eps_ref, z_ref):
    aggm = (jnp.dot(al_ref[...].astype(jnp.float32), wmt_ref[...],
                    preferred_element_type=jnp.float32)
            + jnp.dot(ar_ref[...].astype(jnp.float32), wmb_ref[...],
                      preferred_element_type=jnp.float32))
    h = jnp.maximum(
        jnp.dot(x_ref[...], whx_ref[...], preferred_element_type=jnp.float32)
        + jnp.dot(aggm, wha_ref[...], preferred_element_type=jnp.float32)
        + bh_ref[...], 0.0)
    zm = jnp.dot(h, rmw_ref[...], preferred_element_type=jnp.float32) + rmb_ref[...]
    zlv = -jnp.abs(jnp.dot(h, rvw_ref[...], preferred_element_type=jnp.float32)
                   + rvb_ref[...])
    z_ref[...] = zm + jnp.exp(zlv * 0.5) * eps_ref[...]


_ROWS = 2000


def _dense(x, al, ar, W_m, whx, wha, b_h, R_mean_w, R_mean_b, R_var_w,
           R_var_b, eps):
    grid = (N // _ROWS,)
    full = lambda shape: pl.BlockSpec(shape, lambda i: (0, 0))
    return pl.pallas_call(
        _dense_body,
        grid=grid,
        in_specs=[
            pl.BlockSpec((_ROWS, D), lambda i: (i, 0)),
            pl.BlockSpec((_ROWS, DHALF), lambda i: (i, 0)),
            pl.BlockSpec((_ROWS, DHALF), lambda i: (i, 0)),
            full((DHALF, H)),
            full((DHALF, H)),
            full((D, H)),
            full((H, H)),
            full((1, H)),
            full((H, L)),
            full((1, L)),
            full((H, L)),
            full((1, L)),
            pl.BlockSpec((_ROWS, L), lambda i: (i, 0)),
        ],
        out_specs=pl.BlockSpec((_ROWS, L), lambda i: (i, 0)),
        out_shape=jax.ShapeDtypeStruct((N, L), jnp.float32),
    )(x, al, ar, W_m[:DHALF], W_m[DHALF:], whx, wha, b_h.reshape(1, H),
      R_mean_w, R_mean_b.reshape(1, L), R_var_w, R_var_b.reshape(1, L), eps)


def kernel(x, edge_index, W_m, W_h, b_h, R_mean_w, R_mean_b, R_var_w, R_var_b):
    src = edge_index[0]
    dst = edge_index[1]
    pad = EPAD - E
    src_p = jnp.concatenate([src, jnp.zeros((pad,), jnp.int32)]
                            ).reshape(NS, NCHUNK, CHUNK)
    dst_p = jnp.concatenate([dst, jnp.full((pad,), N, jnp.int32)]
                            ).reshape(NS, NCHUNK, CHUNK)
    xb = x.astype(jnp.bfloat16)
    xl = xb[:, :DHALF]
    xr = xb[:, DHALF:]
    z0 = jnp.zeros((ZROWS, DHALF), jnp.bfloat16)
    aggl, aggr = _sc_agg(xl, xr, src_p, dst_p, z0)
    eps = jax.random.normal(jax.random.key(1), (N, L), dtype=jnp.float32)
    return _dense(x, aggl, aggr, W_m, W_h[:D], W_h[D:], b_h,
                  R_mean_w, R_mean_b, R_var_w, R_var_b, eps)
